# baseline (speedup 1.0000x reference)
import functools

import jax
import jax.numpy as jnp
from jax import lax
from jax.experimental import pallas as pl
from jax.experimental.pallas import tpu as pltpu
from jax.experimental.pallas import tpu_sc as plsc

_NQ = 4
_G = 4
_LANES = 16


def _tbl_body(tok_ref, pos_ref, w_ref, b_ref, tbl_ref):
    vocab, d = tok_ref.shape
    seq = pos_ref.shape[0]
    srel = seq // _NQ
    pos4 = pos_ref[:].reshape(_NQ, srel, d)
    emb = tok_ref[:][None, :, None, :] + pos4[:, None, :, :]
    mean = jnp.mean(emb, axis=-1, keepdims=True)
    var = jnp.mean(jnp.square(emb - mean), axis=-1, keepdims=True)
    normed = (emb - mean) * lax.rsqrt(var + 1e-5)
    tbl = normed * w_ref[:][None, None, None, :] + b_ref[:][None, None, None, :]
    tbl_ref[...] = tbl.reshape(_NQ, vocab * srel * d)


def _build_table_and_base(x, tok_embed, pos_embed, ln_w, ln_b, blocks):
    vocab, d = tok_embed.shape
    batch, seq = x.shape
    srel = seq // _NQ
    tbl = pl.pallas_call(
        _tbl_body,
        out_shape=jax.ShapeDtypeStruct((_NQ, vocab * srel * d), jnp.float32),
    )(tok_embed, pos_embed[:seq], ln_w, ln_b)

    def base_body(x_ref, base_ref):
        xb = x_ref[...]
        smod = lax.broadcasted_iota(jnp.int32, xb.shape, 1) % srel
        base = ((xb * srel + smod) * d).reshape(xb.shape[0], _NQ, srel)
        pad = jnp.zeros((xb.shape[0], _NQ, _LANES - srel), jnp.int32)
        base_ref[...] = jnp.concatenate([base, pad], axis=-1)

    bb = batch // blocks
    base = pl.pallas_call(
        base_body,
        grid=(blocks,),
        in_specs=[pl.BlockSpec((bb, seq), lambda i: (i, 0))],
        out_specs=pl.BlockSpec((bb, _NQ, _LANES), lambda i: (i, 0, 0)),
        out_shape=jax.ShapeDtypeStruct((batch, _NQ, _LANES), jnp.int32),
    )(x)
    base = base.transpose(1, 0, 2).reshape(_NQ, blocks, -1)
    return tbl, base


def _make_sc_build(batch, seq, vocab, d, n_workers):
    srel = seq // _NQ
    row_d = srel * d
    blocks = n_workers // _NQ
    b_per_w = batch // blocks
    half = b_per_w // 2
    gpairs = half // (2 * _G)
    chunks = d // _LANES
    mesh = plsc.VectorSubcoreMesh(core_axis_name="c", subcore_axis_name="s")

    @functools.partial(
        pl.kernel,
        mesh=mesh,
        out_type=jax.ShapeDtypeStruct((batch, _NQ, row_d), jnp.float32),
        scratch_types=[
            pltpu.VMEM((vocab * srel * d,), jnp.float32),
            pltpu.VMEM((half * _LANES,), jnp.int32),
            [pltpu.VMEM((_G, row_d), jnp.float32)] * 2,
            [pltpu.SemaphoreType.DMA] * 2,
        ],
        compiler_params=pltpu.CompilerParams(
            needs_layout_passes=False, use_tc_tiling_on_sc=False),
    )
    def sc_build(tbl_hbm, base_hbm, out_hbm, tbl_l, base_l, stg, wsems):
        n_cores = lax.axis_size("c")
        wid = lax.axis_index("s") * n_cores + lax.axis_index("c")
        q = wid % _NQ
        blk = wid // _NQ
        b0w = blk * b_per_w
        pltpu.sync_copy(tbl_hbm.at[q], tbl_l)
        iota = lax.broadcasted_iota(jnp.int32, (_LANES,), 0)
        offs = [iota + (c * _LANES) for c in range(chunks)]

        splat_dn = lax.GatherDimensionNumbers(
            offset_dims=(), collapsed_slice_dims=(0,), start_index_map=(0,))

        def fill_group(stg_ref, bloc0):
            def bbody(gb, carry):
                brow = bloc0 + gb
                bvec = base_l[pl.ds(brow * _LANES, _LANES)]
                for j in range(srel):
                    bspl = lax.gather(
                        bvec, jnp.full((_LANES, 1), j, jnp.int32), splat_dn,
                        (1,), mode=lax.GatherScatterMode.PROMISE_IN_BOUNDS)
                    for c in range(chunks):
                        stg_ref[gb, pl.ds(j * d + c * _LANES, _LANES)] = (
                            plsc.load_gather(tbl_l, [bspl + offs[c]]))
                return carry

            lax.fori_loop(0, _G, bbody, 0)

        def write_start(p, b0_abs):
            pltpu.async_copy(stg[p], out_hbm.at[pl.ds(b0_abs, _G), q], wsems[p])

        def write_wait(p, b0_abs):
            pltpu.make_async_copy(
                stg[p], out_hbm.at[pl.ds(b0_abs, _G), q], wsems[p]).wait()

        for ph in range(2):
            pb = b0w + ph * half
            pltpu.sync_copy(
                base_hbm.at[q, blk, pl.ds((ph * half) * _LANES, half * _LANES)],
                base_l)

            def gp_body(g2, carry, ph=ph, pb=pb):
                for p in range(2):
                    g = 2 * g2 + p
                    babs = pb + g * _G
                    if ph == 0:
                        @pl.when(g2 > 0)
                        def _():
                            write_wait(p, babs - 2 * _G)
                    else:
                        write_wait(p, babs - 2 * _G)
                    fill_group(stg[p], g * _G)
                    write_start(p, babs)
                return carry

            lax.fori_loop(0, gpairs, gp_body, 0)

        write_wait(0, b0w + b_per_w - 2 * _G)
        write_wait(1, b0w + b_per_w - _G)

    return sc_build


def kernel(x, tok_embed, pos_embed, ln_w, ln_b):
    if x.ndim <= 1:
        x = x.reshape(1, -1)
    batch, seq = x.shape
    vocab, d = tok_embed.shape
    info = plsc.get_sparse_core_info()
    n_workers = info.num_cores * info.num_subcores
    tbl, base = _build_table_and_base(
        x, tok_embed, pos_embed, ln_w, ln_b, n_workers // _NQ)
    out = _make_sc_build(batch, seq, vocab, d, n_workers)(tbl, base)
    return out.reshape(batch, seq, d)


# speedup vs baseline: 1.8964x; 1.8964x over previous
import functools

import jax
import jax.numpy as jnp
from jax import lax
from jax.experimental import pallas as pl
from jax.experimental.pallas import tpu as pltpu
from jax.experimental.pallas import tpu_sc as plsc

_NQ = 4
_G = 4
_LANES = 16


def _tbl_body(tok_ref, pos_ref, w_ref, b_ref, tbl_ref):
    vocab, d = tok_ref.shape
    seq = pos_ref.shape[0]
    srel = seq // _NQ
    pos4 = pos_ref[:].reshape(_NQ, srel, d)
    emb = tok_ref[:][None, :, None, :] + pos4[:, None, :, :]
    mean = jnp.mean(emb, axis=-1, keepdims=True)
    var = jnp.mean(jnp.square(emb - mean), axis=-1, keepdims=True)
    normed = (emb - mean) * lax.rsqrt(var + 1e-5)
    tbl = normed * w_ref[:][None, None, None, :] + b_ref[:][None, None, None, :]
    tbl_ref[...] = tbl.reshape(_NQ, vocab * srel * d)


def _build_table_and_base(x, tok_embed, pos_embed, ln_w, ln_b, blocks):
    vocab, d = tok_embed.shape
    batch, seq = x.shape
    srel = seq // _NQ
    tbl = pl.pallas_call(
        _tbl_body,
        out_shape=jax.ShapeDtypeStruct((_NQ, vocab * srel * d), jnp.float32),
    )(tok_embed, pos_embed[:seq], ln_w, ln_b)

    def base_body(x_ref, base_ref):
        xb = x_ref[...]
        smod = lax.broadcasted_iota(jnp.int32, xb.shape, 1) % srel
        base = ((xb * srel + smod) * d).reshape(xb.shape[0], _NQ, srel)
        pad = jnp.zeros((xb.shape[0], _NQ, _LANES - srel), jnp.int32)
        base_ref[...] = jnp.concatenate([base, pad], axis=-1)

    bb = batch // blocks
    base = pl.pallas_call(
        base_body,
        grid=(blocks,),
        in_specs=[pl.BlockSpec((bb, seq), lambda i: (i, 0))],
        out_specs=pl.BlockSpec((bb, _NQ, _LANES), lambda i: (i, 0, 0)),
        out_shape=jax.ShapeDtypeStruct((batch, _NQ, _LANES), jnp.int32),
    )(x)
    base = base.transpose(1, 0, 2).reshape(_NQ, blocks, -1)
    return tbl, base


def _make_sc_build(batch, seq, vocab, d, n_workers):
    srel = seq // _NQ
    row_d = srel * d
    blocks = n_workers // _NQ
    b_per_w = batch // blocks
    half = b_per_w // 2
    gpairs = half // (2 * _G)
    chunks = d // _LANES
    mesh = plsc.VectorSubcoreMesh(core_axis_name="c", subcore_axis_name="s")

    @functools.partial(
        pl.kernel,
        mesh=mesh,
        out_type=jax.ShapeDtypeStruct((batch, _NQ, row_d), jnp.float32),
        scratch_types=[
            pltpu.VMEM((vocab * srel * d,), jnp.float32),
            pltpu.VMEM((half * _LANES,), jnp.int32),
            [pltpu.VMEM((_G, row_d), jnp.float32)] * 2,
            [pltpu.SemaphoreType.DMA] * 2,
        ],
        compiler_params=pltpu.CompilerParams(
            needs_layout_passes=False, use_tc_tiling_on_sc=False),
    )
    def sc_build(tbl_hbm, base_hbm, out_hbm, tbl_l, base_l, stg, wsems):
        n_cores = lax.axis_size("c")
        wid = lax.axis_index("s") * n_cores + lax.axis_index("c")
        q = wid % _NQ
        blk = wid // _NQ
        b0w = blk * b_per_w
        pltpu.sync_copy(tbl_hbm.at[q], tbl_l)
        iota = lax.broadcasted_iota(jnp.int32, (_LANES,), 0)
        offs = [iota + (c * _LANES) for c in range(chunks)]

        splat_dn = lax.GatherDimensionNumbers(
            offset_dims=(), collapsed_slice_dims=(0,), start_index_map=(0,))

        steps = [(j, c) for j in range(srel) for c in range(chunks)]
        _PRE = 6

        def fill_group(stg_ref, bloc0):
            def bbody(gb, carry):
                brow = bloc0 + gb
                bvec = base_l[pl.ds(brow * _LANES, _LANES)]
                bspls = [
                    lax.gather(
                        bvec, jnp.full((_LANES, 1), j, jnp.int32), splat_dn,
                        (1,), mode=lax.GatherScatterMode.PROMISE_IN_BOUNDS)
                    for j in range(srel)
                ]
                vals = {}
                for k, (j, c) in enumerate(steps):
                    vals[k] = plsc.load_gather(tbl_l, [bspls[j] + offs[c]])
                    if k >= _PRE:
                        jj, cc = steps[k - _PRE]
                        stg_ref[gb, pl.ds(jj * d + cc * _LANES, _LANES)] = (
                            vals.pop(k - _PRE))
                for k in range(len(steps) - _PRE, len(steps)):
                    jj, cc = steps[k]
                    stg_ref[gb, pl.ds(jj * d + cc * _LANES, _LANES)] = vals.pop(k)
                return carry

            lax.fori_loop(0, _G, bbody, 0)

        def write_start(p, b0_abs):
            pltpu.async_copy(stg[p], out_hbm.at[pl.ds(b0_abs, _G), q], wsems[p])

        def write_wait(p, b0_abs):
            pltpu.make_async_copy(
                stg[p], out_hbm.at[pl.ds(b0_abs, _G), q], wsems[p]).wait()

        for ph in range(2):
            pb = b0w + ph * half
            pltpu.sync_copy(
                base_hbm.at[q, blk, pl.ds((ph * half) * _LANES, half * _LANES)],
                base_l)

            def gp_body(g2, carry, ph=ph, pb=pb):
                for p in range(2):
                    g = 2 * g2 + p
                    babs = pb + g * _G
                    if ph == 0:
                        @pl.when(g2 > 0)
                        def _():
                            write_wait(p, babs - 2 * _G)
                    else:
                        write_wait(p, babs - 2 * _G)
                    fill_group(stg[p], g * _G)
                    write_start(p, babs)
                return carry

            lax.fori_loop(0, gpairs, gp_body, 0)

        write_wait(0, b0w + b_per_w - 2 * _G)
        write_wait(1, b0w + b_per_w - _G)

    return sc_build


def kernel(x, tok_embed, pos_embed, ln_w, ln_b):
    if x.ndim <= 1:
        x = x.reshape(1, -1)
    batch, seq = x.shape
    vocab, d = tok_embed.shape
    info = plsc.get_sparse_core_info()
    n_workers = info.num_cores * info.num_subcores
    tbl, base = _build_table_and_base(
        x, tok_embed, pos_embed, ln_w, ln_b, n_workers // _NQ)
    out = _make_sc_build(batch, seq, vocab, d, n_workers)(tbl, base)
    return out.reshape(batch, seq, d)
